# baseline (speedup 1.0000x reference)
import functools

import jax
import jax.numpy as jnp
from jax import lax
from jax.experimental import pallas as pl
from jax.experimental.pallas import tpu as pltpu
from jax.experimental.pallas import tpu_sc as plsc

TSTEPS = 2000
NB = 256
W = 224 * 224
LANES = 16
BR = 8

_mesh = plsc.VectorSubcoreMesh(core_axis_name="c", subcore_axis_name="s")


@functools.partial(
    pl.kernel,
    out_type=[
        jax.ShapeDtypeStruct((NB,), jnp.float32),
        jax.ShapeDtypeStruct((NB,), jnp.float32),
    ],
    mesh=_mesh,
    scratch_types=[
        pltpu.VMEM((LANES,), jnp.int32),
        pltpu.VMEM((LANES,), jnp.float32),
        pltpu.VMEM((LANES,), jnp.float32),
        pltpu.SemaphoreType.DMA,
    ],
)
def _gather_coefs(t_hbm, g_hbm, s_hbm, outg_hbm, outs_hbm,
                  idx_v, gbuf_v, sbuf_v, sem):
    wid = lax.axis_index("s") * 2 + lax.axis_index("c")

    @pl.when(wid < NB // LANES)
    def _():
        base = wid * LANES
        pltpu.sync_copy(t_hbm.at[pl.ds(base, LANES)], idx_v)
        pltpu.async_copy(g_hbm.at[idx_v], gbuf_v, sem).wait()
        pltpu.async_copy(s_hbm.at[idx_v], sbuf_v, sem).wait()
        pltpu.sync_copy(gbuf_v, outg_hbm.at[pl.ds(base, LANES)])
        pltpu.sync_copy(sbuf_v, outs_hbm.at[pl.ds(base, LANES)])


def _scale_add_body(g_ref, s_ref, y_ref, n_ref, oy_ref, on_ref):
    nv = n_ref[...]
    oy_ref[...] = g_ref[...] * y_ref[...] + s_ref[...] * nv
    on_ref[...] = nv


_scale_add_call = pl.pallas_call(
    _scale_add_body,
    grid=(NB // BR,),
    in_specs=[
        pl.BlockSpec((BR, 1), lambda i: (i, 0)),
        pl.BlockSpec((BR, 1), lambda i: (i, 0)),
        pl.BlockSpec((BR, W), lambda i: (i, 0)),
        pl.BlockSpec((BR, W), lambda i: (i, 0)),
    ],
    out_specs=[
        pl.BlockSpec((BR, W), lambda i: (i, 0)),
        pl.BlockSpec((BR, W), lambda i: (i, 0)),
    ],
    out_shape=[
        jax.ShapeDtypeStruct((NB, W), jnp.float32),
        jax.ShapeDtypeStruct((NB, W), jnp.float32),
    ],
)


def kernel(y, noise, t, gammas, sqrt_one_minus_gammas, sqrt_gammas):
    t32 = t.astype(jnp.int32)
    g_t, s_t = _gather_coefs(t32, gammas, sqrt_one_minus_gammas)
    y2 = y.reshape(NB, W)
    n2 = noise.reshape(NB, W)
    oy, on = _scale_add_call(g_t.reshape(NB, 1), s_t.reshape(NB, 1), y2, n2)
    return oy.reshape(y.shape), on.reshape(noise.shape)
